# baseline (speedup 1.0000x reference)
import functools

import jax
import jax.numpy as jnp
from jax.experimental import pallas as pl
from jax.experimental.pallas import tpu as pltpu
from jax.experimental.pallas import tpu_sc as plsc

RANK = 16
EMBED = 64
GROUP = 8
GATHER_WINDOW = 128
G_B = 256


def _sc_gather(table_VE, idx_flat):
    n = idx_flat.shape[0]
    idx2d = idx_flat.reshape(1, n)
    mesh = plsc.VectorSubcoreMesh(core_axis_name="core",
                                  subcore_axis_name="subcore")

    @functools.partial(
        pl.kernel,
        out_type=jax.ShapeDtypeStruct((n, RANK), jnp.float32),
        mesh=mesh,
        compiler_params=pltpu.CompilerParams(use_tc_tiling_on_sc=False),
    )
    def gather_kernel(table_hbm, i_hbm, o_hbm):
        def body(i_vmem, o_vmem):
            pltpu.sync_copy(table_hbm.at[i_vmem.at[0]], o_vmem)

        pltpu.emit_pipeline(
            body,
            grid=(n // GATHER_WINDOW,),
            in_specs=[pl.BlockSpec((1, GATHER_WINDOW),
                                   index_map=lambda i: (0, i))],
            out_specs=[pl.BlockSpec((GATHER_WINDOW, RANK),
                                    index_map=lambda i: (i, 0))],
            core_axis_name=("core", "subcore"),
            dimension_semantics=(pltpu.PARALLEL,),
        )(i_hbm, o_hbm)

    return gather_kernel(table_VE, idx2d)


_VPAD = 1 << 20
_VSTRIDE = _VPAD // 8
_TP_RB = 1024


def _prep_body(*refs):
    (*t_refs, o_ref) = refs
    o_ref[...] = jnp.concatenate([t[...].T for t in t_refs], axis=1)


def _tc_prep_table(table_t):
    last_blk = table_t.shape[1] // _TP_RB
    return pl.pallas_call(
        _prep_body,
        grid=(_VSTRIDE // _TP_RB,),
        in_specs=[
            pl.BlockSpec(
                (RANK, _TP_RB),
                functools.partial(
                    lambda a, j: (0, jnp.minimum(a * 128 + j, last_blk)), a))
            for a in range(8)
        ],
        out_specs=pl.BlockSpec((_TP_RB, 8 * RANK), lambda j: (j, 0)),
        out_shape=jax.ShapeDtypeStruct((_VSTRIDE, 8 * RANK), jnp.float32),
    )(*([table_t] * 8))


def _proj_body(x_ref, w_ref, o_ref, y_scr):
    i = pl.program_id(1)

    @pl.when(i == 0)
    def _():
        y_scr[...] = jax.lax.dot_general(
            w_ref[...], x_ref[...].astype(jnp.bfloat16),
            (((0,), (1,)), ((), ())),
            preferred_element_type=jnp.float32)

    o_ref[...] = y_scr[pl.ds(i * EMBED, EMBED), :]


def _tc_project(x128, w_block, hist, groups_per_l):
    batch = GROUP * groups_per_l
    return pl.pallas_call(
        _proj_body,
        grid=(hist, GROUP),
        in_specs=[
            pl.BlockSpec((groups_per_l, GROUP * RANK), lambda l, i: (l, 0)),
            pl.BlockSpec((GROUP * RANK, GROUP * EMBED), lambda l, i: (0, 0)),
        ],
        out_specs=pl.BlockSpec((EMBED, groups_per_l), lambda l, i: (l, i)),
        out_shape=jax.ShapeDtypeStruct((hist * EMBED, batch), jnp.float32),
        scratch_shapes=[pltpu.VMEM((GROUP * EMBED, groups_per_l),
                                   jnp.float32)],
    )(x128, w_block)


def kernel(token_ids, table_VE, W_EH):
    batch, hist = token_ids.shape
    n = batch * hist
    groups_per_l = batch // GROUP

    idx3 = token_ids.T.reshape(hist, GROUP, groups_per_l)
    idx_perm = idx3.swapaxes(1, 2).reshape(n).astype(jnp.int32)

    table_lin = _tc_prep_table(table_VE.T).reshape(_VPAD, RANK)
    idx_remap = (idx_perm % _VSTRIDE) * 8 + idx_perm // _VSTRIDE

    emb = _sc_gather(table_lin, idx_remap)
    x128 = emb.reshape(n // GROUP, GROUP * RANK)

    w_block = jnp.kron(jnp.eye(GROUP, dtype=jnp.bfloat16),
                       W_EH.T.astype(jnp.bfloat16))

    out = _tc_project(x128, w_block, hist, groups_per_l)
    return out.reshape(hist, EMBED, batch).transpose(2, 0, 1)


# speedup vs baseline: 1.2195x; 1.2195x over previous
import functools

import jax
import jax.numpy as jnp
from jax.experimental import pallas as pl
from jax.experimental.pallas import tpu as pltpu
from jax.experimental.pallas import tpu_sc as plsc

RANK = 16
EMBED = 64
GROUP = 8
GATHER_WINDOW = 128
G_B = 256


def _sc_gather(table_VE, idx_flat):
    n = idx_flat.shape[0]
    idx2d = idx_flat.reshape(1, n)
    mesh = plsc.VectorSubcoreMesh(core_axis_name="core",
                                  subcore_axis_name="subcore")

    @functools.partial(
        pl.kernel,
        out_type=jax.ShapeDtypeStruct((n, RANK), jnp.float32),
        mesh=mesh,
        compiler_params=pltpu.CompilerParams(use_tc_tiling_on_sc=False),
    )
    def gather_kernel(table_hbm, i_hbm, o_hbm):
        def body(i_vmem, o_vmem):
            pltpu.sync_copy(table_hbm.at[i_vmem.at[0]], o_vmem)

        pltpu.emit_pipeline(
            body,
            grid=(n // GATHER_WINDOW,),
            in_specs=[pl.BlockSpec((1, GATHER_WINDOW),
                                   index_map=lambda i: (0, i))],
            out_specs=[pl.BlockSpec((GATHER_WINDOW, RANK),
                                    index_map=lambda i: (i, 0))],
            core_axis_name=("core", "subcore"),
            dimension_semantics=(pltpu.PARALLEL,),
        )(i_hbm, o_hbm)

    return gather_kernel(table_VE, idx2d)


_VPAD = 1 << 20
_VSTRIDE = _VPAD // 8
_TP_RB = 1024


def _prep_body(*refs):
    (*t_refs, o_ref) = refs
    o_ref[...] = jnp.concatenate([t[...] for t in t_refs], axis=0).T


def _tc_prep_table(table_t):
    last_blk = table_t.shape[1] // _TP_RB
    return pl.pallas_call(
        _prep_body,
        grid=(_VSTRIDE // _TP_RB,),
        in_specs=[
            pl.BlockSpec(
                (RANK, _TP_RB),
                functools.partial(
                    lambda a, j: (0, jnp.minimum(a * 128 + j, last_blk)), a))
            for a in range(8)
        ],
        out_specs=pl.BlockSpec((_TP_RB, 8 * RANK), lambda j: (j, 0)),
        out_shape=jax.ShapeDtypeStruct((_VSTRIDE, 8 * RANK), jnp.float32),
    )(*([table_t] * 8))


def _proj_body(x_ref, w_ref, o_ref):
    o_ref[...] = jax.lax.dot_general(
        w_ref[0], x_ref[...].astype(jnp.bfloat16),
        (((0,), (1,)), ((), ())),
        preferred_element_type=jnp.float32)


def _tc_project(x128, w8, hist, groups_per_l):
    batch = GROUP * groups_per_l
    return pl.pallas_call(
        _proj_body,
        grid=(hist, GROUP),
        in_specs=[
            pl.BlockSpec((groups_per_l, GROUP * RANK), lambda l, i: (l, 0)),
            pl.BlockSpec((1, GROUP * RANK, EMBED), lambda l, i: (i, 0, 0)),
        ],
        out_specs=pl.BlockSpec((EMBED, groups_per_l), lambda l, i: (l, i)),
        out_shape=jax.ShapeDtypeStruct((hist * EMBED, batch), jnp.float32),
    )(x128, w8)


def kernel(token_ids, table_VE, W_EH):
    batch, hist = token_ids.shape
    n = batch * hist
    groups_per_l = batch // GROUP

    idx3 = token_ids.T.reshape(hist, GROUP, groups_per_l)
    idx_perm = idx3.swapaxes(1, 2).reshape(n).astype(jnp.int32)

    table_lin = _tc_prep_table(table_VE.T).reshape(_VPAD, RANK)
    idx_remap = (idx_perm % _VSTRIDE) * 8 + idx_perm // _VSTRIDE

    emb = _sc_gather(table_lin, idx_remap)
    x128 = emb.reshape(n // GROUP, GROUP * RANK)

    w_block = jnp.kron(jnp.eye(GROUP, dtype=jnp.bfloat16),
                       W_EH.T.astype(jnp.bfloat16))
    w8 = w_block.reshape(GROUP * RANK, GROUP, EMBED).swapaxes(0, 1)

    out = _tc_project(x128, w8, hist, groups_per_l)
    return out.reshape(hist, EMBED, batch).transpose(2, 0, 1)


# speedup vs baseline: 1.3978x; 1.1463x over previous
import functools

import jax
import jax.numpy as jnp
from jax.experimental import pallas as pl
from jax.experimental.pallas import tpu as pltpu
from jax.experimental.pallas import tpu_sc as plsc

RANK = 16
EMBED = 64
GROUP = 8
GATHER_WINDOW = 512
G_B = 256


def _sc_gather(table_VE, idx_flat):
    n = idx_flat.shape[0]
    idx2d = idx_flat.reshape(1, n)
    mesh = plsc.VectorSubcoreMesh(core_axis_name="core",
                                  subcore_axis_name="subcore")

    @functools.partial(
        pl.kernel,
        out_type=jax.ShapeDtypeStruct((n, RANK), jnp.float32),
        mesh=mesh,
        compiler_params=pltpu.CompilerParams(use_tc_tiling_on_sc=False),
    )
    def gather_kernel(table_hbm, i_hbm, o_hbm):
        def body(i_vmem, o_vmem):
            pltpu.sync_copy(table_hbm.at[i_vmem.at[0]], o_vmem)

        pltpu.emit_pipeline(
            body,
            grid=(n // GATHER_WINDOW,),
            in_specs=[pl.BlockSpec((1, GATHER_WINDOW),
                                   index_map=lambda i: (0, i))],
            out_specs=[pl.BlockSpec((GATHER_WINDOW, RANK),
                                    index_map=lambda i: (i, 0))],
            core_axis_name=("core", "subcore"),
            dimension_semantics=(pltpu.PARALLEL,),
        )(i_hbm, o_hbm)

    return gather_kernel(table_VE, idx2d)


_VPAD = 1 << 20
_VSTRIDE = _VPAD // 8
_TP_RB = 1024


def _prep_body(*refs):
    (*t_refs, o_ref) = refs
    o_ref[...] = jnp.concatenate([t[...] for t in t_refs], axis=0).T


def _tc_prep_table(table_t):
    last_blk = table_t.shape[1] // _TP_RB
    return pl.pallas_call(
        _prep_body,
        grid=(_VSTRIDE // _TP_RB,),
        in_specs=[
            pl.BlockSpec(
                (RANK, _TP_RB),
                functools.partial(
                    lambda a, j: (0, jnp.minimum(a * 128 + j, last_blk)), a))
            for a in range(8)
        ],
        out_specs=pl.BlockSpec((_TP_RB, 8 * RANK), lambda j: (j, 0)),
        out_shape=jax.ShapeDtypeStruct((_VSTRIDE, 8 * RANK), jnp.float32),
    )(*([table_t] * 8))


def _proj_body(x_ref, w_ref, o_ref):
    o_ref[...] = jax.lax.dot_general(
        w_ref[0], x_ref[...].astype(jnp.bfloat16),
        (((0,), (1,)), ((), ())),
        preferred_element_type=jnp.float32)


def _tc_project(x128, w8, hist, groups_per_l):
    batch = GROUP * groups_per_l
    return pl.pallas_call(
        _proj_body,
        grid=(hist, GROUP),
        in_specs=[
            pl.BlockSpec((groups_per_l, GROUP * RANK), lambda l, i: (l, 0)),
            pl.BlockSpec((1, GROUP * RANK, EMBED), lambda l, i: (i, 0, 0)),
        ],
        out_specs=pl.BlockSpec((EMBED, groups_per_l), lambda l, i: (l, i)),
        out_shape=jax.ShapeDtypeStruct((hist * EMBED, batch), jnp.float32),
    )(x128, w8)


def kernel(token_ids, table_VE, W_EH):
    batch, hist = token_ids.shape
    n = batch * hist
    groups_per_l = batch // GROUP

    idx3 = token_ids.T.reshape(hist, GROUP, groups_per_l)
    idx_perm = idx3.swapaxes(1, 2).reshape(n).astype(jnp.int32)

    table_lin = _tc_prep_table(table_VE.T).reshape(_VPAD, RANK)
    idx_remap = (idx_perm % _VSTRIDE) * 8 + idx_perm // _VSTRIDE

    emb = _sc_gather(table_lin, idx_remap)
    x128 = emb.reshape(n // GROUP, GROUP * RANK)

    w_block = jnp.kron(jnp.eye(GROUP, dtype=jnp.bfloat16),
                       W_EH.T.astype(jnp.bfloat16))
    w8 = w_block.reshape(GROUP * RANK, GROUP, EMBED).swapaxes(0, 1)

    out = _tc_project(x128, w8, hist, groups_per_l)
    return out.reshape(hist, EMBED, batch).transpose(2, 0, 1)
